# Initial kernel scaffold; baseline (speedup 1.0000x reference)
import jax
import jax.numpy as jnp
from jax.experimental import pallas as pl


def kernel(tx_x, entity_x, edge_index, emb_tables, tx_w, tx_b, gat_w, att_src, att_dst, gat_b, ln_g, ln_b, w1, b1, w2, b2, w3, b3):
    raise NotImplementedError("write your pallas kernel here")


# speedup vs baseline: 11.0106x; 11.0106x over previous
import functools

import jax
import jax.numpy as jnp
from jax import lax
from jax.experimental import pallas as pl
from jax.experimental.pallas import tpu as pltpu
from jax.experimental.pallas import tpu_sc as plsc

T = 11
D = 128
E = 320000
NTX = 10000
NENT = 10000
VOCAB = 10000
EPS = 1e-5

APL = 10240
APP = APL + 8
CH = 80
EPT2 = E // 16
EPT3 = E // 32
SB = 4000
NCH = SB // CH
GRP = 10
RB = 1000
RBE = 1024
TV = T * VOCAB
GROWS = 2 * TV
EXPAD = 240


def _prek_body(emb_ref, w_ref, av_ref, dv_ref, g_ref, as_ref, ad_ref):
    b = pl.program_id(2)
    tw = jnp.dot(emb_ref[0], w_ref[...], preferred_element_type=jnp.float32)
    as_row = jnp.sum(tw * av_ref[...], axis=1)
    ad_row = jnp.sum(tw * dv_ref[...], axis=1)
    scale = jnp.where(b == 0, 1.0, 0.2)
    e = jnp.exp(scale * as_row)
    g_ref[0, 0] = tw * e[:, None]
    as_ref[0, 0] = as_row.reshape(RB, 1)
    ad_ref[0, 0] = ad_row.reshape(RB, 1)


def _build_tables(emb_tables, gat_w, att_src, att_dst):
    grid = (T, VOCAB // RB, 2)
    return pl.pallas_call(
        _prek_body,
        grid=grid,
        in_specs=[
            pl.BlockSpec((1, RB, D), lambda t, i, b: (t, i, 0)),
            pl.BlockSpec((D, D), lambda t, i, b: (0, 0)),
            pl.BlockSpec((1, D), lambda t, i, b: (0, 0)),
            pl.BlockSpec((1, D), lambda t, i, b: (0, 0)),
        ],
        out_specs=[
            pl.BlockSpec((1, 1, RB, D), lambda t, i, b: (b, t, i, 0)),
            pl.BlockSpec((1, 1, RB, 1), lambda t, i, b: (t, i, 0, 0)),
            pl.BlockSpec((1, 1, RB, 1), lambda t, i, b: (t, i, 0, 0)),
        ],
        out_shape=[
            jax.ShapeDtypeStruct((2, T, VOCAB, D), jnp.float32),
            jax.ShapeDtypeStruct((T, VOCAB // RB, RB, 1), jnp.float32),
            jax.ShapeDtypeStruct((T, VOCAB // RB, RB, 1), jnp.float32),
        ],
    )(emb_tables, gat_w, att_src.reshape(1, D), att_dst.reshape(1, D))


def _sc_edge_body(srcx_hbm, dstx_hbm, ex2_hbm, as_hbm, ad_hbm, g_hbm, vacc,
                  sacc, adn, srcb, dstb, s2b, d2b, fsb, fdb, idxg, idxs, idxd,
                  dnb, rows, zbuf, zs, exn, adnb, acc, accs, sem, sem2):
    c = lax.axis_index("c")
    s = lax.axis_index("s")

    zero16 = jnp.zeros((16,), jnp.float32)

    def zfill(i, _):
        r = i // 8
        j = i % 8
        zbuf[r, pl.ds(j * 16, 16)] = zero16
        return 0

    lax.fori_loop(0, 64 * 8, zfill, 0)

    def zsfill(i, _):
        zs[pl.ds(i * 16, 16)] = zero16
        return 0

    lax.fori_loop(0, 40, zsfill, 0)

    scale = jnp.where(c == 0, 1.0, 0.2)

    def t_body(t, _):
        def zc(q, _):
            pltpu.sync_copy(zbuf, acc.at[pl.ds(s * 640 + q * 64, 64)])
            return 0

        lax.fori_loop(0, 10, zc, 0)
        pltpu.sync_copy(zs, accs.at[pl.ds(s * 640, 640)])
        plsc.subcore_barrier()

        def sub_body(bi, _):
            ebase = t * E + s * EPT2 + bi * SB
            pltpu.sync_copy(srcx_hbm.at[pl.ds(ebase, SB)], srcb)
            pltpu.sync_copy(dstx_hbm.at[pl.ds(ebase, SB)], dstb)

            def g1(g, _):
                def fire(ci, _):
                    o = (g * GRP + ci) * CH
                    pltpu.async_copy(ex2_hbm.at[srcb.at[pl.ds(o, CH)]],
                                     s2b.at[pl.ds(o, CH)], sem)
                    pltpu.async_copy(ex2_hbm.at[dstb.at[pl.ds(o, CH)]],
                                     d2b.at[pl.ds(o, CH)], sem)
                    return 0

                def drain(ci, _):
                    o = (g * GRP + ci) * CH
                    pltpu.make_async_copy(ex2_hbm.at[srcb.at[pl.ds(o, CH)]],
                                          s2b.at[pl.ds(o, CH)], sem).wait()
                    pltpu.make_async_copy(ex2_hbm.at[dstb.at[pl.ds(o, CH)]],
                                          d2b.at[pl.ds(o, CH)], sem).wait()
                    return 0

                lax.fori_loop(0, GRP, fire, 0)
                lax.fori_loop(0, GRP, drain, 0)
                return 0

            lax.fori_loop(0, NCH // GRP, g1, 0)

            def g2(g, _):
                def fire(ci, _):
                    o = (g * GRP + ci) * CH
                    pltpu.async_copy(as_hbm.at[s2b.at[pl.ds(o, CH)]],
                                     fsb.at[pl.ds(o, CH)], sem)
                    pltpu.async_copy(ad_hbm.at[d2b.at[pl.ds(o, CH)]],
                                     fdb.at[pl.ds(o, CH)], sem)
                    return 0

                def drain(ci, _):
                    o = (g * GRP + ci) * CH
                    pltpu.make_async_copy(as_hbm.at[s2b.at[pl.ds(o, CH)]],
                                          fsb.at[pl.ds(o, CH)], sem).wait()
                    pltpu.make_async_copy(ad_hbm.at[d2b.at[pl.ds(o, CH)]],
                                          fdb.at[pl.ds(o, CH)], sem).wait()
                    return 0

                lax.fori_loop(0, GRP, fire, 0)
                lax.fori_loop(0, GRP, drain, 0)
                return 0

            lax.fori_loop(0, NCH // GRP, g2, 0)

            def chunk(ci, _):
                for v in range(5):
                    off = ci * CH + v * 16
                    fs = fsb[pl.ds(off, 16)]
                    a = fs + fdb[pl.ds(off, 16)]
                    br = jnp.where(a > 0.0, 0, 1)
                    idxg[pl.ds(v * 16, 16)] = s2b[pl.ds(off, 16)] + c * TV
                    dv = dstb[pl.ds(off, 16)] - t * NENT
                    dr = jnp.where(br == c, dv, APL)
                    idxs[pl.ds(v * 16, 16)] = dr
                    idxd[pl.ds(v * 16, 16)] = dr
                    dnb[pl.ds(v * 16, 16)] = jnp.exp(scale * fs)
                pltpu.async_copy(g_hbm.at[idxg], rows, sem2).wait()
                pltpu.sync_copy(rows, acc.at[idxs], add=True)
                pltpu.sync_copy(dnb, accs.at[idxd], add=True)
                return 0

            lax.fori_loop(0, NCH, chunk, 0)
            return 0

        lax.fori_loop(0, EPT2 // SB, sub_body, 0)

        @pl.when(c == 0)
        def _():
            nb = s * 640
            pltpu.sync_copy(ex2_hbm.at[pl.ds(t * NENT + nb, 640)], exn)

            def af(k, _):
                o = k * CH
                pltpu.async_copy(ad_hbm.at[exn.at[pl.ds(o, CH)]],
                                 adnb.at[pl.ds(o, CH)], sem)
                return 0

            def ad_drain(k, _):
                o = k * CH
                pltpu.make_async_copy(ad_hbm.at[exn.at[pl.ds(o, CH)]],
                                      adnb.at[pl.ds(o, CH)], sem).wait()
                return 0

            lax.fori_loop(0, 640 // CH, af, 0)
            lax.fori_loop(0, 640 // CH, ad_drain, 0)
            pltpu.sync_copy(adnb, adn.at[pl.ds(t * APL + nb, 640)])

        plsc.subcore_barrier()

        def wb(q, _):
            r = s * 640 + q * 128
            pltpu.sync_copy(acc.at[pl.ds(r, 128)], vacc.at[c, t, pl.ds(r, 128)])
            return 0

        lax.fori_loop(0, 5, wb, 0)
        pltpu.sync_copy(accs.at[pl.ds(s * 640, 640)],
                        sacc.at[c, t, pl.ds(s * 640, 640)])
        return 0

    lax.fori_loop(0, T, t_body, 0)


def _sc_edge(srcx_flat, dstx_flat, ex2_flat, as_flat, ad_flat, g_flat):
    mesh = plsc.VectorSubcoreMesh(core_axis_name="c", subcore_axis_name="s")
    f = functools.partial(
        pl.kernel,
        out_type=(
            jax.ShapeDtypeStruct((2, T, APL, D), jnp.float32),
            jax.ShapeDtypeStruct((2, T, APL), jnp.float32),
            jax.ShapeDtypeStruct((T * APL,), jnp.float32),
        ),
        mesh=mesh,
        scratch_types=[
            pltpu.VMEM((SB,), jnp.int32),
            pltpu.VMEM((SB,), jnp.int32),
            pltpu.VMEM((SB,), jnp.int32),
            pltpu.VMEM((SB,), jnp.int32),
            pltpu.VMEM((SB,), jnp.float32),
            pltpu.VMEM((SB,), jnp.float32),
            pltpu.VMEM((CH,), jnp.int32),
            pltpu.VMEM((CH,), jnp.int32),
            pltpu.VMEM((CH,), jnp.int32),
            pltpu.VMEM((CH,), jnp.float32),
            pltpu.VMEM((CH, D), jnp.float32),
            pltpu.VMEM((64, D), jnp.float32),
            pltpu.VMEM((640,), jnp.float32),
            pltpu.VMEM((640,), jnp.int32),
            pltpu.VMEM((640,), jnp.float32),
            pltpu.VMEM_SHARED((APP, D), jnp.float32),
            pltpu.VMEM_SHARED((APP,), jnp.float32),
            pltpu.SemaphoreType.DMA,
            pltpu.SemaphoreType.DMA,
        ],
    )(_sc_edge_body)
    return f(srcx_flat, dstx_flat, ex2_flat, as_flat, ad_flat, g_flat)


def _epi_body(vp_ref, vm_ref, sp_ref, sm_ref, adn_ref, gb_ref, lg_ref, lb_ref,
              h_ref):
    vp = vp_ref[0, 0]
    vm = vm_ref[0, 0]
    sp = sp_ref[0, 0]
    sm = sm_ref[0, 0]
    adnv = adn_ref[0]
    fp = jnp.exp(adnv)
    fm = jnp.exp(0.2 * adnv)
    num = fp * vp + fm * vm
    den = fp * sp + fm * sm + 1e-16
    o = num / den + gb_ref[...]
    mean = jnp.mean(o, axis=1, keepdims=True)
    var = jnp.mean((o - mean) ** 2, axis=1, keepdims=True)
    h = (o - mean) / jnp.sqrt(var + EPS) * lg_ref[0] + lb_ref[0]
    h_ref[0] = h


def _epilogue(vacc, sacc, adn, gat_b, ln_g, ln_b):
    grid = (T, APL // RBE)
    sacc3 = sacc.reshape(2, T, APL, 1)
    return pl.pallas_call(
        _epi_body,
        grid=grid,
        in_specs=[
            pl.BlockSpec((1, 1, RBE, D), lambda t, i: (0, t, i, 0)),
            pl.BlockSpec((1, 1, RBE, D), lambda t, i: (1, t, i, 0)),
            pl.BlockSpec((1, 1, RBE, 1), lambda t, i: (0, t, i, 0)),
            pl.BlockSpec((1, 1, RBE, 1), lambda t, i: (1, t, i, 0)),
            pl.BlockSpec((1, RBE, 1), lambda t, i: (t, i, 0)),
            pl.BlockSpec((1, D), lambda t, i: (0, 0)),
            pl.BlockSpec((1, 1, D), lambda t, i: (t, 0, 0)),
            pl.BlockSpec((1, 1, D), lambda t, i: (t, 0, 0)),
        ],
        out_specs=pl.BlockSpec((1, RBE, D), lambda t, i: (t, i, 0)),
        out_shape=jax.ShapeDtypeStruct((T, APL, D), jnp.float32),
    )(vacc, vacc, sacc3, sacc3, adn.reshape(T, APL, 1), gat_b.reshape(1, D),
      ln_g.reshape(T, 1, D), ln_b.reshape(T, 1, D))


def _sc_agg_body(src_hbm, dst_hbm, h_hbm, agg, srcb, dstb, idxg, idxs, rows,
                 zbuf, acc, sem):
    c = lax.axis_index("c")
    s = lax.axis_index("s")
    zero16 = jnp.zeros((16,), jnp.float32)

    def zfill(i, _):
        r = i // 8
        j = i % 8
        zbuf[r, pl.ds(j * 16, 16)] = zero16
        return 0

    lax.fori_loop(0, 128 * 8, zfill, 0)

    def t_body(t, _):
        ebase = t * E + c * (E // 2) + s * EPT3
        pltpu.sync_copy(src_hbm.at[pl.ds(ebase, EPT3)], srcb)
        pltpu.sync_copy(dst_hbm.at[pl.ds(ebase, EPT3)], dstb)

        def zc(q, _):
            pltpu.sync_copy(zbuf, acc.at[pl.ds(s * 640 + q * 128, 128)])
            return 0

        lax.fori_loop(0, 5, zc, 0)
        plsc.subcore_barrier()

        def chunk(ci, _):
            for v in range(5):
                off = ci * CH + v * 16
                sv = srcb[pl.ds(off, 16)]
                dv = dstb[pl.ds(off, 16)]
                idxg[pl.ds(v * 16, 16)] = sv + t * APL
                idxs[pl.ds(v * 16, 16)] = dv
            pltpu.async_copy(h_hbm.at[idxg], rows, sem).wait()
            pltpu.sync_copy(rows, acc.at[idxs], add=True)
            return 0

        lax.fori_loop(0, EPT3 // CH, chunk, 0)
        plsc.subcore_barrier()

        def wb(q, _):
            r = s * 640 + q * 128
            pltpu.sync_copy(acc.at[pl.ds(r, 128)], agg.at[c, t, pl.ds(r, 128)])
            return 0

        lax.fori_loop(0, 5, wb, 0)
        return 0

    lax.fori_loop(0, T, t_body, 0)


def _sc_agg(src_flat, dst_flat, h_flat):
    mesh = plsc.VectorSubcoreMesh(core_axis_name="c", subcore_axis_name="s")
    f = functools.partial(
        pl.kernel,
        out_type=jax.ShapeDtypeStruct((2, T, APL, D), jnp.float32),
        mesh=mesh,
        scratch_types=[
            pltpu.VMEM((EPT3,), jnp.int32),
            pltpu.VMEM((EPT3,), jnp.int32),
            pltpu.VMEM((CH,), jnp.int32),
            pltpu.VMEM((CH,), jnp.int32),
            pltpu.VMEM((CH, D), jnp.float32),
            pltpu.VMEM((128, D), jnp.float32),
            pltpu.VMEM_SHARED((APL, D), jnp.float32),
            pltpu.SemaphoreType.DMA,
        ],
    )(_sc_agg_body)
    return f(src_flat, dst_flat, h_flat)


def _mlp_body(tx_ref, agg_ref, w1_ref, b1_ref, w2_ref, b2_ref, w3_ref, b3_ref,
              out_ref):
    acc = jnp.dot(tx_ref[...], w1_ref[0], preferred_element_type=jnp.float32)
    for t in range(T):
        m = agg_ref[0, t] + agg_ref[1, t]
        acc = acc + jnp.dot(m, w1_ref[t + 1], preferred_element_type=jnp.float32)
    h1 = jnp.maximum(acc + b1_ref[...], 0.0)
    h2 = jnp.maximum(
        jnp.dot(h1, w2_ref[...], preferred_element_type=jnp.float32)
        + b2_ref[...], 0.0)
    out_ref[...] = (
        jnp.dot(h2, w3_ref[...], preferred_element_type=jnp.float32)
        + b3_ref[...])


def _mlp(tx_x, agg, w1, b1, w2, b2, w3, b3):
    grid = (NTX // RB,)
    return pl.pallas_call(
        _mlp_body,
        grid=grid,
        in_specs=[
            pl.BlockSpec((RB, D), lambda i: (i, 0)),
            pl.BlockSpec((2, T, RB, D), lambda i: (0, 0, i, 0)),
            pl.BlockSpec((T + 1, D, D), lambda i: (0, 0, 0)),
            pl.BlockSpec((1, D), lambda i: (0, 0)),
            pl.BlockSpec((D, 64), lambda i: (0, 0)),
            pl.BlockSpec((1, 64), lambda i: (0, 0)),
            pl.BlockSpec((64, 1), lambda i: (0, 0)),
            pl.BlockSpec((1, 1), lambda i: (0, 0)),
        ],
        out_specs=pl.BlockSpec((RB, 1), lambda i: (i, 0)),
        out_shape=jax.ShapeDtypeStruct((NTX, 1), jnp.float32),
    )(tx_x, agg, w1.reshape(T + 1, D, D), b1.reshape(1, D), w2,
      b2.reshape(1, 64), w3, b3.reshape(1, 1))


def kernel(tx_x, entity_x, edge_index, emb_tables, tx_w, tx_b, gat_w,
           att_src, att_dst, gat_b, ln_g, ln_b, w1, b1, w2, b2, w3, b3):
    g4, as4, ad4 = _build_tables(emb_tables, gat_w, att_src, att_dst)
    g_flat = g4.reshape(GROWS, D)
    as_flat = as4.reshape(TV)
    ad_flat = ad4.reshape(TV)
    edge_index = edge_index.astype(jnp.int32)
    src_flat = edge_index[:, 0, :].reshape(T * E)
    dst_flat = edge_index[:, 1, :].reshape(T * E)
    toff = jnp.arange(T, dtype=jnp.int32)[:, None]
    srcx_flat = (edge_index[:, 0, :] + toff * NENT).reshape(T * E)
    dstx_flat = (edge_index[:, 1, :] + toff * NENT).reshape(T * E)
    ex2 = entity_x.astype(jnp.int32) + toff * VOCAB
    ex2_flat = jnp.pad(ex2.reshape(T * NENT), (0, EXPAD))
    vacc, sacc, adn = _sc_edge(srcx_flat, dstx_flat, ex2_flat, as_flat,
                               ad_flat, g_flat)
    h = _epilogue(vacc, sacc, adn, gat_b, ln_g, ln_b)
    h_flat = h.reshape(T * APL, D)
    agg = _sc_agg(src_flat, dst_flat, h_flat)
    return _mlp(tx_x, agg, w1, b1, w2, b2, w3, b3)


# speedup vs baseline: 11.9281x; 1.0833x over previous
import functools

import jax
import jax.numpy as jnp
from jax import lax
from jax.experimental import pallas as pl
from jax.experimental.pallas import tpu as pltpu
from jax.experimental.pallas import tpu_sc as plsc

T = 11
D = 128
E = 320000
NTX = 10000
NENT = 10000
VOCAB = 10000
EPS = 1e-5

APL = 10240
APP = APL + 8
CH = 80
TV = T * VOCAB
GR = 2 * TV
TNP2 = 112640
NPS = TNP2 // 16
NGRP = 8
EPSC = E // 16
SB = 2000
NCHE = SB // CH
GRPE = 5
EPT3 = E // 32
RB = 1000
RBE = 1024


def _prek_body(emb_ref, w_ref, av_ref, dv_ref, g_ref, as_ref, ad_ref):
    b = pl.program_id(2)
    tw = jnp.dot(emb_ref[0], w_ref[...], preferred_element_type=jnp.float32)
    as_row = jnp.sum(tw * av_ref[...], axis=1)
    ad_row = jnp.sum(tw * dv_ref[...], axis=1)
    scale = jnp.where(b == 0, 1.0, 0.2)
    e = jnp.exp(scale * as_row)
    g_ref[0, 0] = tw * e[:, None]
    as_ref[0, 0] = as_row.reshape(RB, 1)
    ad_ref[0, 0] = ad_row.reshape(RB, 1)


def _build_tables(emb_tables, gat_w, att_src, att_dst):
    grid = (T, VOCAB // RB, 2)
    return pl.pallas_call(
        _prek_body,
        grid=grid,
        in_specs=[
            pl.BlockSpec((1, RB, D), lambda t, i, b: (t, i, 0)),
            pl.BlockSpec((D, D), lambda t, i, b: (0, 0)),
            pl.BlockSpec((1, D), lambda t, i, b: (0, 0)),
            pl.BlockSpec((1, D), lambda t, i, b: (0, 0)),
        ],
        out_specs=[
            pl.BlockSpec((1, 1, RB, D), lambda t, i, b: (b, t, i, 0)),
            pl.BlockSpec((1, 1, RB, 1), lambda t, i, b: (t, i, 0, 0)),
            pl.BlockSpec((1, 1, RB, 1), lambda t, i, b: (t, i, 0, 0)),
        ],
        out_shape=[
            jax.ShapeDtypeStruct((2, T, VOCAB, D), jnp.float32),
            jax.ShapeDtypeStruct((T, VOCAB // RB, RB, 1), jnp.float32),
            jax.ShapeDtypeStruct((T, VOCAB // RB, RB, 1), jnp.float32),
        ],
    )(emb_tables, gat_w, att_src.reshape(1, D), att_dst.reshape(1, D))


def _sc_nodes_body(ex2_hbm, as_hbm, ad_hbm, asn, adn, adapl, exb, vb, exn,
                   adnb, sem):
    c = lax.axis_index("c")
    s = lax.axis_index("s")
    base = s * NPS
    pltpu.sync_copy(ex2_hbm.at[pl.ds(base, NPS)], exb)

    def gather_plane(tbl, out):
        def g(gi, _):
            def fire(ci, _):
                o = (gi * NGRP + ci) * CH
                pltpu.async_copy(tbl.at[exb.at[pl.ds(o, CH)]],
                                 vb.at[pl.ds(o, CH)], sem)
                return 0

            def drain(ci, _):
                o = (gi * NGRP + ci) * CH
                pltpu.make_async_copy(tbl.at[exb.at[pl.ds(o, CH)]],
                                      vb.at[pl.ds(o, CH)], sem).wait()
                return 0

            lax.fori_loop(0, NGRP, fire, 0)
            lax.fori_loop(0, NGRP, drain, 0)
            return 0

        lax.fori_loop(0, (NPS // CH) // NGRP, g, 0)
        pltpu.sync_copy(vb, out.at[pl.ds(base, NPS)])

    @pl.when(c == 0)
    def _():
        gather_plane(as_hbm, asn)

        def t_ex(t, _):
            nb = s * 640
            pltpu.sync_copy(ex2_hbm.at[pl.ds(t * NENT + nb, 640)], exn)

            def af(k, _):
                o = k * CH
                pltpu.async_copy(ad_hbm.at[exn.at[pl.ds(o, CH)]],
                                 adnb.at[pl.ds(o, CH)], sem)
                return 0

            def adr(k, _):
                o = k * CH
                pltpu.make_async_copy(ad_hbm.at[exn.at[pl.ds(o, CH)]],
                                      adnb.at[pl.ds(o, CH)], sem).wait()
                return 0

            lax.fori_loop(0, 640 // CH, af, 0)
            lax.fori_loop(0, 640 // CH, adr, 0)
            pltpu.sync_copy(adnb, adapl.at[pl.ds(t * APL + nb, 640)])
            return 0

        lax.fori_loop(0, T, t_ex, 0)

    @pl.when(c == 1)
    def _():
        gather_plane(ad_hbm, adn)


def _sc_nodes(ex2_flat, as_flat, ad_flat):
    mesh = plsc.VectorSubcoreMesh(core_axis_name="c", subcore_axis_name="s")
    f = functools.partial(
        pl.kernel,
        out_type=(
            jax.ShapeDtypeStruct((TNP2,), jnp.float32),
            jax.ShapeDtypeStruct((TNP2,), jnp.float32),
            jax.ShapeDtypeStruct((T * APL,), jnp.float32),
        ),
        mesh=mesh,
        scratch_types=[
            pltpu.VMEM((NPS,), jnp.int32),
            pltpu.VMEM((NPS,), jnp.float32),
            pltpu.VMEM((640,), jnp.int32),
            pltpu.VMEM((640,), jnp.float32),
            pltpu.SemaphoreType.DMA,
        ],
    )(_sc_nodes_body)
    return f(ex2_flat, as_flat, ad_flat)


def _sc_edge_body(srcx_hbm, dstx_hbm, ex2_hbm, asn_hbm, adn_hbm, g_hbm, vacc,
                  sacc, srcb, dstb, s2b, asb, adb, idxg, idxs, dnb, rows,
                  zbuf, zs, acc, accs, sem, sem2):
    c = lax.axis_index("c")
    s = lax.axis_index("s")

    zero16 = jnp.zeros((16,), jnp.float32)

    def zfill(i, _):
        r = i // 8
        j = i % 8
        zbuf[r, pl.ds(j * 16, 16)] = zero16
        return 0

    lax.fori_loop(0, 64 * 8, zfill, 0)

    def zsfill(i, _):
        zs[pl.ds(i * 16, 16)] = zero16
        return 0

    lax.fori_loop(0, 40, zsfill, 0)

    scale = jnp.where(c == 0, 1.0, 0.2)

    def t_body(t, _):
        def zc(q, _):
            pltpu.sync_copy(zbuf, acc.at[pl.ds(s * 640 + q * 64, 64)])
            return 0

        lax.fori_loop(0, 10, zc, 0)
        pltpu.sync_copy(zs, accs.at[pl.ds(s * 640, 640)])
        plsc.subcore_barrier()

        def sub_body(bi, _):
            ebase = t * E + s * EPSC + bi * SB
            pltpu.sync_copy(srcx_hbm.at[pl.ds(ebase, SB)], srcb)
            pltpu.sync_copy(dstx_hbm.at[pl.ds(ebase, SB)], dstb)

            def g1(g, _):
                def fire(ci, _):
                    o = (g * GRPE + ci) * CH
                    pltpu.async_copy(ex2_hbm.at[srcb.at[pl.ds(o, CH)]],
                                     s2b.at[pl.ds(o, CH)], sem)
                    pltpu.async_copy(asn_hbm.at[srcb.at[pl.ds(o, CH)]],
                                     asb.at[pl.ds(o, CH)], sem)
                    pltpu.async_copy(adn_hbm.at[dstb.at[pl.ds(o, CH)]],
                                     adb.at[pl.ds(o, CH)], sem)
                    return 0

                def drain(ci, _):
                    o = (g * GRPE + ci) * CH
                    pltpu.make_async_copy(ex2_hbm.at[srcb.at[pl.ds(o, CH)]],
                                          s2b.at[pl.ds(o, CH)], sem).wait()
                    pltpu.make_async_copy(asn_hbm.at[srcb.at[pl.ds(o, CH)]],
                                          asb.at[pl.ds(o, CH)], sem).wait()
                    pltpu.make_async_copy(adn_hbm.at[dstb.at[pl.ds(o, CH)]],
                                          adb.at[pl.ds(o, CH)], sem).wait()
                    return 0

                lax.fori_loop(0, GRPE, fire, 0)
                lax.fori_loop(0, GRPE, drain, 0)
                return 0

            lax.fori_loop(0, NCHE // GRPE, g1, 0)

            def chunk(ci, _):
                for v in range(5):
                    off = ci * CH + v * 16
                    fs = asb[pl.ds(off, 16)]
                    a = fs + adb[pl.ds(off, 16)]
                    br = jnp.where(a > 0.0, 0, 1)
                    idxg[pl.ds(v * 16, 16)] = s2b[pl.ds(off, 16)] + c * TV
                    dv = dstb[pl.ds(off, 16)] - t * NENT
                    idxs[pl.ds(v * 16, 16)] = jnp.where(br == c, dv, APL)
                    dnb[pl.ds(v * 16, 16)] = jnp.exp(scale * fs)
                pltpu.async_copy(g_hbm.at[idxg], rows, sem2).wait()
                pltpu.sync_copy(rows, acc.at[idxs], add=True)
                pltpu.sync_copy(dnb, accs.at[idxs], add=True)
                return 0

            lax.fori_loop(0, NCHE, chunk, 0)
            return 0

        lax.fori_loop(0, EPSC // SB, sub_body, 0)
        plsc.subcore_barrier()

        def wb(q, _):
            r = s * 640 + q * 128
            pltpu.sync_copy(acc.at[pl.ds(r, 128)], vacc.at[c, t, pl.ds(r, 128)])
            return 0

        lax.fori_loop(0, 5, wb, 0)
        pltpu.sync_copy(accs.at[pl.ds(s * 640, 640)],
                        sacc.at[c, t, pl.ds(s * 640, 640)])
        return 0

    lax.fori_loop(0, T, t_body, 0)


def _sc_edge(srcx_flat, dstx_flat, ex2_flat, asn, adn, g_flat):
    mesh = plsc.VectorSubcoreMesh(core_axis_name="c", subcore_axis_name="s")
    f = functools.partial(
        pl.kernel,
        out_type=(
            jax.ShapeDtypeStruct((2, T, APL, D), jnp.float32),
            jax.ShapeDtypeStruct((2, T, APL), jnp.float32),
        ),
        mesh=mesh,
        scratch_types=[
            pltpu.VMEM((SB,), jnp.int32),
            pltpu.VMEM((SB,), jnp.int32),
            pltpu.VMEM((SB,), jnp.int32),
            pltpu.VMEM((SB,), jnp.float32),
            pltpu.VMEM((SB,), jnp.float32),
            pltpu.VMEM((CH,), jnp.int32),
            pltpu.VMEM((CH,), jnp.int32),
            pltpu.VMEM((CH,), jnp.float32),
            pltpu.VMEM((CH, D), jnp.float32),
            pltpu.VMEM((64, D), jnp.float32),
            pltpu.VMEM((640,), jnp.float32),
            pltpu.VMEM_SHARED((APP, D), jnp.float32),
            pltpu.VMEM_SHARED((APP,), jnp.float32),
            pltpu.SemaphoreType.DMA,
            pltpu.SemaphoreType.DMA,
        ],
    )(_sc_edge_body)
    return f(srcx_flat, dstx_flat, ex2_flat, asn, adn, g_flat)


def _epi_body(vp_ref, vm_ref, sp_ref, sm_ref, adn_ref, gb_ref, lg_ref, lb_ref,
              h_ref):
    vp = vp_ref[0, 0]
    vm = vm_ref[0, 0]
    sp = sp_ref[0, 0]
    sm = sm_ref[0, 0]
    adnv = adn_ref[0]
    fp = jnp.exp(adnv)
    fm = jnp.exp(0.2 * adnv)
    num = fp * vp + fm * vm
    den = fp * sp + fm * sm + 1e-16
    o = num / den + gb_ref[...]
    mean = jnp.mean(o, axis=1, keepdims=True)
    var = jnp.mean((o - mean) ** 2, axis=1, keepdims=True)
    h = (o - mean) / jnp.sqrt(var + EPS) * lg_ref[0] + lb_ref[0]
    h_ref[0] = h


def _epilogue(vacc, sacc, adapl, gat_b, ln_g, ln_b):
    grid = (T, APL // RBE)
    sacc3 = sacc.reshape(2, T, APL, 1)
    return pl.pallas_call(
        _epi_body,
        grid=grid,
        in_specs=[
            pl.BlockSpec((1, 1, RBE, D), lambda t, i: (0, t, i, 0)),
            pl.BlockSpec((1, 1, RBE, D), lambda t, i: (1, t, i, 0)),
            pl.BlockSpec((1, 1, RBE, 1), lambda t, i: (0, t, i, 0)),
            pl.BlockSpec((1, 1, RBE, 1), lambda t, i: (1, t, i, 0)),
            pl.BlockSpec((1, RBE, 1), lambda t, i: (t, i, 0)),
            pl.BlockSpec((1, D), lambda t, i: (0, 0)),
            pl.BlockSpec((1, 1, D), lambda t, i: (t, 0, 0)),
            pl.BlockSpec((1, 1, D), lambda t, i: (t, 0, 0)),
        ],
        out_specs=pl.BlockSpec((1, RBE, D), lambda t, i: (t, i, 0)),
        out_shape=jax.ShapeDtypeStruct((T, APL, D), jnp.float32),
    )(vacc, vacc, sacc3, sacc3, adapl.reshape(T, APL, 1),
      gat_b.reshape(1, D), ln_g.reshape(T, 1, D), ln_b.reshape(T, 1, D))


def _sc_agg_body(src_hbm, dst_hbm, h_hbm, agg, srcb, dstb, idxg, idxs, rows,
                 zbuf, acc, sem):
    c = lax.axis_index("c")
    s = lax.axis_index("s")
    zero16 = jnp.zeros((16,), jnp.float32)

    def zfill(i, _):
        r = i // 8
        j = i % 8
        zbuf[r, pl.ds(j * 16, 16)] = zero16
        return 0

    lax.fori_loop(0, 128 * 8, zfill, 0)

    def t_body(t, _):
        ebase = t * E + c * (E // 2) + s * EPT3
        pltpu.sync_copy(src_hbm.at[pl.ds(ebase, EPT3)], srcb)
        pltpu.sync_copy(dst_hbm.at[pl.ds(ebase, EPT3)], dstb)

        def zc(q, _):
            pltpu.sync_copy(zbuf, acc.at[pl.ds(s * 640 + q * 128, 128)])
            return 0

        lax.fori_loop(0, 5, zc, 0)
        plsc.subcore_barrier()

        def chunk(ci, _):
            for v in range(5):
                off = ci * CH + v * 16
                sv = srcb[pl.ds(off, 16)]
                dv = dstb[pl.ds(off, 16)]
                idxg[pl.ds(v * 16, 16)] = sv + t * APL
                idxs[pl.ds(v * 16, 16)] = dv
            pltpu.async_copy(h_hbm.at[idxg], rows, sem).wait()
            pltpu.sync_copy(rows, acc.at[idxs], add=True)
            return 0

        lax.fori_loop(0, EPT3 // CH, chunk, 0)
        plsc.subcore_barrier()

        def wb(q, _):
            r = s * 640 + q * 128
            pltpu.sync_copy(acc.at[pl.ds(r, 128)], agg.at[c, t, pl.ds(r, 128)])
            return 0

        lax.fori_loop(0, 5, wb, 0)
        return 0

    lax.fori_loop(0, T, t_body, 0)


def _sc_agg(src_flat, dst_flat, h_flat):
    mesh = plsc.VectorSubcoreMesh(core_axis_name="c", subcore_axis_name="s")
    f = functools.partial(
        pl.kernel,
        out_type=jax.ShapeDtypeStruct((2, T, APL, D), jnp.float32),
        mesh=mesh,
        scratch_types=[
            pltpu.VMEM((EPT3,), jnp.int32),
            pltpu.VMEM((EPT3,), jnp.int32),
            pltpu.VMEM((CH,), jnp.int32),
            pltpu.VMEM((CH,), jnp.int32),
            pltpu.VMEM((CH, D), jnp.float32),
            pltpu.VMEM((128, D), jnp.float32),
            pltpu.VMEM_SHARED((APL, D), jnp.float32),
            pltpu.SemaphoreType.DMA,
        ],
    )(_sc_agg_body)
    return f(src_flat, dst_flat, h_flat)


def _mlp_body(tx_ref, agg_ref, w1_ref, b1_ref, w2_ref, b2_ref, w3_ref, b3_ref,
              out_ref):
    acc = jnp.dot(tx_ref[...], w1_ref[0], preferred_element_type=jnp.float32)
    for t in range(T):
        m = agg_ref[0, t] + agg_ref[1, t]
        acc = acc + jnp.dot(m, w1_ref[t + 1], preferred_element_type=jnp.float32)
    h1 = jnp.maximum(acc + b1_ref[...], 0.0)
    h2 = jnp.maximum(
        jnp.dot(h1, w2_ref[...], preferred_element_type=jnp.float32)
        + b2_ref[...], 0.0)
    out_ref[...] = (
        jnp.dot(h2, w3_ref[...], preferred_element_type=jnp.float32)
        + b3_ref[...])


def _mlp(tx_x, agg, w1, b1, w2, b2, w3, b3):
    grid = (NTX // RB,)
    return pl.pallas_call(
        _mlp_body,
        grid=grid,
        in_specs=[
            pl.BlockSpec((RB, D), lambda i: (i, 0)),
            pl.BlockSpec((2, T, RB, D), lambda i: (0, 0, i, 0)),
            pl.BlockSpec((T + 1, D, D), lambda i: (0, 0, 0)),
            pl.BlockSpec((1, D), lambda i: (0, 0)),
            pl.BlockSpec((D, 64), lambda i: (0, 0)),
            pl.BlockSpec((1, 64), lambda i: (0, 0)),
            pl.BlockSpec((64, 1), lambda i: (0, 0)),
            pl.BlockSpec((1, 1), lambda i: (0, 0)),
        ],
        out_specs=pl.BlockSpec((RB, 1), lambda i: (i, 0)),
        out_shape=jax.ShapeDtypeStruct((NTX, 1), jnp.float32),
    )(tx_x, agg, w1.reshape(T + 1, D, D), b1.reshape(1, D), w2,
      b2.reshape(1, 64), w3, b3.reshape(1, 1))


def kernel(tx_x, entity_x, edge_index, emb_tables, tx_w, tx_b, gat_w,
           att_src, att_dst, gat_b, ln_g, ln_b, w1, b1, w2, b2, w3, b3):
    g4, as4, ad4 = _build_tables(emb_tables, gat_w, att_src, att_dst)
    g_flat = g4.reshape(GR, D)
    as_flat = as4.reshape(TV)
    ad_flat = ad4.reshape(TV)
    edge_index = edge_index.astype(jnp.int32)
    src_flat = edge_index[:, 0, :].reshape(T * E)
    dst_flat = edge_index[:, 1, :].reshape(T * E)
    toff = jnp.arange(T, dtype=jnp.int32)[:, None]
    srcx_flat = (edge_index[:, 0, :] + toff * NENT).reshape(T * E)
    dstx_flat = (edge_index[:, 1, :] + toff * NENT).reshape(T * E)
    ex2 = entity_x.astype(jnp.int32) + toff * VOCAB
    ex2_flat = jnp.pad(ex2.reshape(T * NENT), (0, TNP2 - T * NENT))
    asn, adn, adapl = _sc_nodes(ex2_flat, as_flat, ad_flat)
    vacc, sacc = _sc_edge(srcx_flat, dstx_flat, ex2_flat, asn, adn, g_flat)
    h = _epilogue(vacc, sacc, adapl, gat_b, ln_g, ln_b)
    h_flat = h.reshape(T * APL, D)
    agg = _sc_agg(src_flat, dst_flat, h_flat)
    return _mlp(tx_x, agg, w1, b1, w2, b2, w3, b3)
